# baseline (device time: 110454 ns/iter reference)
import os

import jax
import jax.numpy as jnp
from jax import lax
from jax.experimental import pallas as pl
from jax.experimental.pallas import tpu as pltpu

_MODE = os.environ.get("KERNEL_MODE", "full")

H, S, D = 16, 1024, 128
HD = H * D
SCALE = D ** -0.5
PH = H // 2
PW = PH * D
NC = 4
CH = PH // NC
CW = CH * D


def kernel(Q, K, V):
    q32 = Q.reshape(S, HD)
    k32 = K.reshape(S, HD)
    v32 = V.reshape(S, HD)

    def body(q32_ref, k32_ref, v32_ref, o_ref,
             q_ref, k_ref, v_ref, kr_ref, vr_ref, ab_ref, ar_ref,
             l_ref, lb_ref, lr_ref,
             ky_send, vy_send, k_recv, v_recv,
             aa_send, ll_send, a_recv, l_recv):
        ix = lax.axis_index("x")
        iy = lax.axis_index("y")
        iz = lax.axis_index("z")
        ynbr = (ix, 1 - iy, iz)
        xnbr = (1 - ix, iy, iz)

        do_comm = _MODE in ("full", "comm")
        do_compute = _MODE in ("full", "compute")

        barrier = pltpu.get_barrier_semaphore()
        for nbr in (ynbr, xnbr):
            pl.semaphore_signal(
                barrier, inc=1, device_id=nbr,
                device_id_type=pl.DeviceIdType.MESH,
            )
        pl.semaphore_wait(barrier, 2)

        ones = jnp.ones((S,), jnp.bfloat16)

        psl = pl.ds(ix * PW, PW)
        osl = pl.ds((1 - ix) * PW, PW)
        k_ref[:, psl] = k32_ref[:, psl].astype(jnp.bfloat16)
        v_ref[:, psl] = v32_ref[:, psl].astype(jnp.bfloat16)

        yks, yvs = [], []
        for c in range(NC) if do_comm else ():
            asl = pl.ds(ix * PW + c * CW, CW)
            csl = pl.ds(c * CW, CW)
            yk = pltpu.make_async_remote_copy(
                src_ref=k_ref.at[:, asl], dst_ref=kr_ref.at[:, csl],
                send_sem=ky_send.at[c], recv_sem=k_recv.at[c],
                device_id=ynbr, device_id_type=pl.DeviceIdType.MESH,
            )
            yv = pltpu.make_async_remote_copy(
                src_ref=v_ref.at[:, asl], dst_ref=vr_ref.at[:, csl],
                send_sem=vy_send.at[c], recv_sem=v_recv.at[c],
                device_id=ynbr, device_id_type=pl.DeviceIdType.MESH,
            )
            yk.start()
            yv.start()
            yks.append(yk)
            yvs.append(yv)

        k_ref[:, osl] = k32_ref[:, osl].astype(jnp.bfloat16)
        v_ref[:, osl] = v32_ref[:, osl].astype(jnp.bfloat16)
        q_ref[...] = (q32_ref[...] * SCALE).astype(jnp.bfloat16)

        def local_head(h, carry):
            sl = pl.ds(pl.multiple_of(h * D, D), D)
            s = lax.dot_general(
                q_ref[:, sl], k_ref[:, sl], (((1,), (1,)), ((), ())),
                preferred_element_type=jnp.float32,
            )
            p = jnp.exp(s).astype(jnp.bfloat16)
            l_ref[h, :] = lax.dot_general(
                p, ones, (((1,), (0,)), ((), ())),
                preferred_element_type=jnp.float32,
            )
            o_ref[:, sl] = lax.dot_general(
                p, v_ref[:, sl], (((1,), (0,)), ((), ())),
                preferred_element_type=jnp.float32,
            )
            return carry

        if do_compute:
            lax.fori_loop(0, H, local_head, 0)

        acs, lcs = [], []
        for c in range(NC):
            if do_comm:
                yks[c].wait_recv()
                yvs[c].wait_recv()
            elif do_compute:
                kr_ref[:, pl.ds(c * CW, CW)] = k_ref[:, pl.ds(ix * PW + c * CW, CW)]
                vr_ref[:, pl.ds(c * CW, CW)] = v_ref[:, pl.ds(ix * PW + c * CW, CW)]

            def cross_head(j, carry, c=c):
                hc = c * CH + j
                habs = ix * PH + hc
                asl = pl.ds(pl.multiple_of(habs * D, D), D)
                csl = pl.ds(pl.multiple_of(hc * D, D), D)
                s = lax.dot_general(
                    q_ref[:, asl], kr_ref[:, csl], (((1,), (1,)), ((), ())),
                    preferred_element_type=jnp.float32,
                )
                p = jnp.exp(s).astype(jnp.bfloat16)
                l2 = lax.dot_general(
                    p, ones, (((1,), (0,)), ((), ())),
                    preferred_element_type=jnp.float32,
                )
                lb_ref[hc, :] = l2
                acc2 = lax.dot_general(
                    p, vr_ref[:, csl], (((1,), (0,)), ((), ())),
                    preferred_element_type=jnp.float32,
                )
                ab_ref[:, csl] = acc2.astype(jnp.bfloat16)
                inv = 1.0 / (l_ref[habs, :] + l2)
                o_ref[:, asl] = (o_ref[:, asl] + acc2) * inv[:, None]
                return carry

            if do_compute:
                lax.fori_loop(0, CH, cross_head, 0)

            if do_comm:
                csl = pl.ds(c * CW, CW)
                rsl = pl.ds(c * CH, CH)
                ac = pltpu.make_async_remote_copy(
                    src_ref=ab_ref.at[:, csl], dst_ref=ar_ref.at[:, csl],
                    send_sem=aa_send.at[c], recv_sem=a_recv.at[c],
                    device_id=xnbr, device_id_type=pl.DeviceIdType.MESH,
                )
                lc = pltpu.make_async_remote_copy(
                    src_ref=lb_ref.at[rsl, :], dst_ref=lr_ref.at[rsl, :],
                    send_sem=ll_send.at[c], recv_sem=l_recv.at[c],
                    device_id=xnbr, device_id_type=pl.DeviceIdType.MESH,
                )
                ac.start()
                lc.start()
                acs.append(ac)
                lcs.append(lc)
            elif do_compute:
                ar_ref[:, pl.ds(c * CW, CW)] = ab_ref[:, pl.ds(c * CW, CW)]
                lr_ref[c * CH: (c + 1) * CH, :] = lb_ref[c * CH: (c + 1) * CH, :]

        for c in range(NC):
            if do_comm:
                acs[c].wait_recv()
                lcs[c].wait_recv()

            def merge_head(j, carry, c=c):
                hc = c * CH + j
                habs = (1 - ix) * PH + hc
                asl = pl.ds(pl.multiple_of(habs * D, D), D)
                csl = pl.ds(pl.multiple_of(hc * D, D), D)
                l2 = lr_ref[hc, :]
                inv = 1.0 / (l_ref[habs, :] + l2)
                o_ref[:, asl] = (
                    o_ref[:, asl] + ar_ref[:, csl].astype(jnp.float32)
                ) * inv[:, None]
                return carry

            if do_compute:
                lax.fori_loop(0, CH, merge_head, 0)

        if not do_compute:
            o_ref[...] = jnp.zeros((S, HD), jnp.float32)

        for c in range(NC) if do_comm else ():
            yks[c].wait_send()
            yvs[c].wait_send()
            acs[c].wait_send()
            lcs[c].wait_send()

    o2 = pl.pallas_call(
        body,
        out_shape=jax.ShapeDtypeStruct((S, HD), jnp.float32),
        in_specs=[pl.BlockSpec(memory_space=pltpu.VMEM)] * 3,
        out_specs=pl.BlockSpec(memory_space=pltpu.VMEM),
        scratch_shapes=[
            pltpu.VMEM((S, HD), jnp.bfloat16),
            pltpu.VMEM((S, HD), jnp.bfloat16),
            pltpu.VMEM((S, HD), jnp.bfloat16),
            pltpu.VMEM((S, PW), jnp.bfloat16),
            pltpu.VMEM((S, PW), jnp.bfloat16),
            pltpu.VMEM((S, PW), jnp.bfloat16),
            pltpu.VMEM((S, PW), jnp.bfloat16),
            pltpu.VMEM((H, S), jnp.float32),
            pltpu.VMEM((PH, S), jnp.float32),
            pltpu.VMEM((PH, S), jnp.float32),
            pltpu.SemaphoreType.DMA((NC,)),
            pltpu.SemaphoreType.DMA((NC,)),
            pltpu.SemaphoreType.DMA((NC,)),
            pltpu.SemaphoreType.DMA((NC,)),
            pltpu.SemaphoreType.DMA((NC,)),
            pltpu.SemaphoreType.DMA((NC,)),
            pltpu.SemaphoreType.DMA((NC,)),
            pltpu.SemaphoreType.DMA((NC,)),
        ],
        compiler_params=pltpu.CompilerParams(
            collective_id=0, vmem_limit_bytes=100 * 1024 * 1024,
        ),
    )(q32, k32, v32)
    return o2.reshape(1, S, H, D)


# device time: 82805 ns/iter; 1.3339x vs baseline; 1.3339x over previous
import os

import jax
import jax.numpy as jnp
from jax import lax
from jax.experimental import pallas as pl
from jax.experimental.pallas import tpu as pltpu

_MODE = os.environ.get("KERNEL_MODE", "full")

H, S, D = 16, 1024, 128
HD = H * D
SCALE = D ** -0.5
PH = H // 2
PW = PH * D
NC = 8
CH = PH // NC
CW = CH * D


def kernel(Q, K, V):
    q2 = (Q * SCALE).reshape(S, HD).astype(jnp.bfloat16)
    k2 = K.reshape(S, HD).astype(jnp.bfloat16)
    v2 = V.reshape(S, HD).astype(jnp.bfloat16)

    def body(q_ref, k_ref, v_ref, o_ref, kr_ref, vr_ref, ab_ref, ar_ref,
             l_ref, lb_ref, lr_ref,
             ky_send, vy_send, k_recv, v_recv,
             aa_send, ll_send, a_recv, l_recv):
        ix = lax.axis_index("x")
        iy = lax.axis_index("y")
        iz = lax.axis_index("z")
        ynbr = (ix, 1 - iy, iz)
        xnbr = (1 - ix, iy, iz)

        do_comm = _MODE in ("full", "comm")
        do_compute = _MODE in ("full", "compute")

        barrier = pltpu.get_barrier_semaphore()
        for nbr in (ynbr, xnbr):
            pl.semaphore_signal(
                barrier, inc=1, device_id=nbr,
                device_id_type=pl.DeviceIdType.MESH,
            )
        pl.semaphore_wait(barrier, 2)

        ones = jnp.ones((S,), jnp.bfloat16)

        yks, yvs = [], []
        for c in range(NC) if do_comm else ():
            asl = pl.ds(ix * PW + c * CW, CW)
            csl = pl.ds(c * CW, CW)
            yk = pltpu.make_async_remote_copy(
                src_ref=k_ref.at[:, asl], dst_ref=kr_ref.at[:, csl],
                send_sem=ky_send.at[c], recv_sem=k_recv.at[c],
                device_id=ynbr, device_id_type=pl.DeviceIdType.MESH,
            )
            yv = pltpu.make_async_remote_copy(
                src_ref=v_ref.at[:, asl], dst_ref=vr_ref.at[:, csl],
                send_sem=vy_send.at[c], recv_sem=v_recv.at[c],
                device_id=ynbr, device_id_type=pl.DeviceIdType.MESH,
            )
            yk.start()
            yv.start()
            yks.append(yk)
            yvs.append(yv)

        def local_head(h, carry):
            sl = pl.ds(pl.multiple_of(h * D, D), D)
            s = lax.dot_general(
                q_ref[:, sl], k_ref[:, sl], (((1,), (1,)), ((), ())),
                preferred_element_type=jnp.float32,
            )
            p = jnp.exp(s).astype(jnp.bfloat16)
            l_ref[h, :] = lax.dot_general(
                p, ones, (((1,), (0,)), ((), ())),
                preferred_element_type=jnp.float32,
            )
            o_ref[:, sl] = lax.dot_general(
                p, v_ref[:, sl], (((1,), (0,)), ((), ())),
                preferred_element_type=jnp.float32,
            )
            return carry

        if do_compute:
            lax.fori_loop(0, H, local_head, 0)

        acs, lcs = [], []
        for c in range(NC):
            if do_comm:
                yks[c].wait_recv()
                yvs[c].wait_recv()
            elif do_compute:
                kr_ref[:, pl.ds(c * CW, CW)] = k_ref[:, pl.ds(ix * PW + c * CW, CW)]
                vr_ref[:, pl.ds(c * CW, CW)] = v_ref[:, pl.ds(ix * PW + c * CW, CW)]

            def cross_head(j, carry, c=c):
                hc = c * CH + j
                habs = ix * PH + hc
                asl = pl.ds(pl.multiple_of(habs * D, D), D)
                csl = pl.ds(pl.multiple_of(hc * D, D), D)
                s = lax.dot_general(
                    q_ref[:, asl], kr_ref[:, csl], (((1,), (1,)), ((), ())),
                    preferred_element_type=jnp.float32,
                )
                p = jnp.exp(s).astype(jnp.bfloat16)
                l2 = lax.dot_general(
                    p, ones, (((1,), (0,)), ((), ())),
                    preferred_element_type=jnp.float32,
                )
                lb_ref[hc, :] = l2
                acc2 = lax.dot_general(
                    p, vr_ref[:, csl], (((1,), (0,)), ((), ())),
                    preferred_element_type=jnp.float32,
                )
                ab_ref[:, csl] = acc2.astype(jnp.bfloat16)
                inv = 1.0 / (l_ref[habs, :] + l2)
                o_ref[:, asl] = (o_ref[:, asl] + acc2) * inv[:, None]
                return carry

            if do_compute:
                lax.fori_loop(0, CH, cross_head, 0)

            if do_comm:
                csl = pl.ds(c * CW, CW)
                rsl = pl.ds(c * CH, CH)
                ac = pltpu.make_async_remote_copy(
                    src_ref=ab_ref.at[:, csl], dst_ref=ar_ref.at[:, csl],
                    send_sem=aa_send.at[c], recv_sem=a_recv.at[c],
                    device_id=xnbr, device_id_type=pl.DeviceIdType.MESH,
                )
                lc = pltpu.make_async_remote_copy(
                    src_ref=lb_ref.at[rsl, :], dst_ref=lr_ref.at[rsl, :],
                    send_sem=ll_send.at[c], recv_sem=l_recv.at[c],
                    device_id=xnbr, device_id_type=pl.DeviceIdType.MESH,
                )
                ac.start()
                lc.start()
                acs.append(ac)
                lcs.append(lc)
            elif do_compute:
                ar_ref[:, pl.ds(c * CW, CW)] = ab_ref[:, pl.ds(c * CW, CW)]
                lr_ref[c * CH: (c + 1) * CH, :] = lb_ref[c * CH: (c + 1) * CH, :]

        for c in range(NC):
            if do_comm:
                acs[c].wait_recv()
                lcs[c].wait_recv()

            def merge_head(j, carry, c=c):
                hc = c * CH + j
                habs = (1 - ix) * PH + hc
                asl = pl.ds(pl.multiple_of(habs * D, D), D)
                csl = pl.ds(pl.multiple_of(hc * D, D), D)
                l2 = lr_ref[hc, :]
                inv = 1.0 / (l_ref[habs, :] + l2)
                o_ref[:, asl] = (
                    o_ref[:, asl] + ar_ref[:, csl].astype(jnp.float32)
                ) * inv[:, None]
                return carry

            if do_compute:
                lax.fori_loop(0, CH, merge_head, 0)

        if not do_compute:
            o_ref[...] = jnp.zeros((S, HD), jnp.float32)

        for c in range(NC) if do_comm else ():
            yks[c].wait_send()
            yvs[c].wait_send()
            acs[c].wait_send()
            lcs[c].wait_send()

    o2 = pl.pallas_call(
        body,
        out_shape=jax.ShapeDtypeStruct((S, HD), jnp.float32),
        in_specs=[pl.BlockSpec(memory_space=pltpu.VMEM)] * 3,
        out_specs=pl.BlockSpec(memory_space=pltpu.VMEM),
        scratch_shapes=[
            pltpu.VMEM((S, PW), jnp.bfloat16),
            pltpu.VMEM((S, PW), jnp.bfloat16),
            pltpu.VMEM((S, PW), jnp.bfloat16),
            pltpu.VMEM((S, PW), jnp.bfloat16),
            pltpu.VMEM((H, S), jnp.float32),
            pltpu.VMEM((PH, S), jnp.float32),
            pltpu.VMEM((PH, S), jnp.float32),
            pltpu.SemaphoreType.DMA((NC,)),
            pltpu.SemaphoreType.DMA((NC,)),
            pltpu.SemaphoreType.DMA((NC,)),
            pltpu.SemaphoreType.DMA((NC,)),
            pltpu.SemaphoreType.DMA((NC,)),
            pltpu.SemaphoreType.DMA((NC,)),
            pltpu.SemaphoreType.DMA((NC,)),
            pltpu.SemaphoreType.DMA((NC,)),
        ],
        compiler_params=pltpu.CompilerParams(collective_id=0),
    )(q2, k2, v2)
    return o2.reshape(1, S, H, D)
